# baseline (speedup 1.0000x reference)
import functools

import jax
import jax.numpy as jnp
from jax import lax
from jax.experimental import pallas as pl
from jax.experimental.pallas import tpu as pltpu
from jax.experimental.pallas import tpu_sc as plsc

B = 4096
T = 200
D_OUT = 112
BT = B * T
NW = 32
CH = 128
N_CH = BT // CH
CH_PER_W = N_CH // NW


def _sc_assemble(didx2d, s0r2d, s1r2d, bidx2d, sreal, dyn_real2d,
                 t0, t1, dt):
  mesh = plsc.VectorSubcoreMesh(core_axis_name="c", subcore_axis_name="s")

  @functools.partial(
      pl.kernel,
      out_type=jax.ShapeDtypeStruct((BT, D_OUT), jnp.float32),
      mesh=mesh,
      compiler_params=pltpu.CompilerParams(use_tc_tiling_on_sc=False),
      scratch_types=[
          pltpu.VMEM((CH,), jnp.int32),
          pltpu.VMEM((CH,), jnp.int32),
          pltpu.VMEM((CH,), jnp.int32),
          pltpu.VMEM((CH,), jnp.int32),
          pltpu.VMEM((CH, 32), jnp.float32),
          pltpu.VMEM((CH, 32), jnp.float32),
          pltpu.VMEM((CH, 32), jnp.float32),
          pltpu.VMEM((CH, 8), jnp.float32),
          pltpu.VMEM((CH, 8), jnp.float32),
          pltpu.SemaphoreType.DMA,
          pltpu.SemaphoreType.DMA,
      ],
  )
  def k(didx_hbm, s0r_hbm, s1r_hbm, bidx_hbm, sreal_hbm, dr_hbm,
        t0_hbm, t1_hbm, dt_hbm, out_hbm,
        didx_v, s0i_v, s1i_v, bidx_v, drows_v, s0rows_v, s1rows_v,
        srrows_v, dreal_v, sem_in, sem_out):
    cid = lax.axis_index("c")
    sid = lax.axis_index("s")
    wid = sid * 2 + cid
    c0 = wid * CH_PER_W

    def body(g, carry):
      r0 = (c0 + g) * CH
      st = [
          pltpu.async_copy(didx_hbm.at[c0 + g], didx_v, sem_in),
          pltpu.async_copy(s0r_hbm.at[c0 + g], s0i_v, sem_in),
          pltpu.async_copy(s1r_hbm.at[c0 + g], s1i_v, sem_in),
          pltpu.async_copy(bidx_hbm.at[c0 + g], bidx_v, sem_in),
          pltpu.async_copy(dr_hbm.at[pl.ds(r0, CH)], dreal_v, sem_in),
      ]
      for cp in st:
        cp.wait()
      gs = [
          pltpu.async_copy(dt_hbm.at[didx_v], drows_v, sem_in),
          pltpu.async_copy(t0_hbm.at[s0i_v], s0rows_v, sem_in),
          pltpu.async_copy(t1_hbm.at[s1i_v], s1rows_v, sem_in),
          pltpu.async_copy(sreal_hbm.at[bidx_v], srrows_v, sem_in),
      ]
      for cp in gs:
        cp.wait()
      ws = [
          pltpu.async_copy(s0rows_v,
                           out_hbm.at[pl.ds(r0, CH), pl.ds(0, 32)], sem_out),
          pltpu.async_copy(s1rows_v,
                           out_hbm.at[pl.ds(r0, CH), pl.ds(32, 32)], sem_out),
          pltpu.async_copy(srrows_v,
                           out_hbm.at[pl.ds(r0, CH), pl.ds(64, 8)], sem_out),
          pltpu.async_copy(drows_v,
                           out_hbm.at[pl.ds(r0, CH), pl.ds(72, 32)], sem_out),
          pltpu.async_copy(dreal_v,
                           out_hbm.at[pl.ds(r0, CH), pl.ds(104, 8)], sem_out),
      ]
      for cp in ws:
        cp.wait()
      return carry

    lax.fori_loop(0, CH_PER_W, body, 0)

  return k(didx2d, s0r2d, s1r2d, bidx2d, sreal, dyn_real2d, t0, t1, dt)


def kernel(feat_static_cat, feat_static_real, feat_dynamic_cat,
           feat_dynamic_real, static_table0, static_table1, dyn_table0):
  didx2d = feat_dynamic_cat.astype(jnp.int32).reshape(N_CH, CH)
  s0r2d = jnp.repeat(feat_static_cat[:, 0].astype(jnp.int32),
                     T).reshape(N_CH, CH)
  s1r2d = jnp.repeat(feat_static_cat[:, 1].astype(jnp.int32),
                     T).reshape(N_CH, CH)
  bidx2d = jnp.repeat(jnp.arange(B, dtype=jnp.int32), T).reshape(N_CH, CH)
  dr2d = feat_dynamic_real.reshape(BT, 8)
  out = _sc_assemble(didx2d, s0r2d, s1r2d, bidx2d, feat_static_real, dr2d,
                     static_table0, static_table1, dyn_table0)
  return out.reshape(B, T, D_OUT)


# speedup vs baseline: 1.1909x; 1.1909x over previous
import functools

import jax
import jax.numpy as jnp
from jax import lax
from jax.experimental import pallas as pl
from jax.experimental.pallas import tpu as pltpu
from jax.experimental.pallas import tpu_sc as plsc

B = 4096
T = 200
D_OUT = 112
BT = B * T
NW = 32
L = 128
SPC = 2
CH = L * SPC
N_L = BT // L
N_CH = BT // CH
CH_PER_W = N_CH // NW


def _sc_assemble(didx2d, s0r2d, s1r2d, bidx2d, sreal, dyn_real2d,
                 t0, t1, dt):
  mesh = plsc.VectorSubcoreMesh(core_axis_name="c", subcore_axis_name="s")

  @functools.partial(
      pl.kernel,
      out_type=jax.ShapeDtypeStruct((BT, D_OUT), jnp.float32),
      mesh=mesh,
      compiler_params=pltpu.CompilerParams(use_tc_tiling_on_sc=False),
      scratch_types=[
          pltpu.VMEM((2, SPC, L), jnp.int32),
          pltpu.VMEM((2, SPC, L), jnp.int32),
          pltpu.VMEM((2, SPC, L), jnp.int32),
          pltpu.VMEM((2, SPC, L), jnp.int32),
          pltpu.VMEM((2, CH, 32), jnp.float32),
          pltpu.VMEM((2, CH, 32), jnp.float32),
          pltpu.VMEM((2, CH, 32), jnp.float32),
          pltpu.VMEM((2, CH, 8), jnp.float32),
          pltpu.VMEM((2, CH, 8), jnp.float32),
          pltpu.SemaphoreType.DMA,
          pltpu.SemaphoreType.DMA,
          pltpu.SemaphoreType.DMA,
      ],
  )
  def k(didx_hbm, s0r_hbm, s1r_hbm, bidx_hbm, sreal_hbm, dr_hbm,
        t0_hbm, t1_hbm, dt_hbm, out_hbm,
        didx_v, s0i_v, s1i_v, bidx_v, drows_v, s0rows_v, s1rows_v,
        srrows_v, dreal_v, sem_st, sem_g, sem_w):
    cid = lax.axis_index("c")
    sid = lax.axis_index("s")
    wid = sid * 2 + cid
    c0 = wid * CH_PER_W

    def a_pairs(g, s):
      q0 = (c0 + g) * SPC
      r0 = (c0 + g) * CH
      return [
          (didx_hbm.at[pl.ds(q0, SPC)], didx_v.at[s]),
          (s0r_hbm.at[pl.ds(q0, SPC)], s0i_v.at[s]),
          (s1r_hbm.at[pl.ds(q0, SPC)], s1i_v.at[s]),
          (bidx_hbm.at[pl.ds(q0, SPC)], bidx_v.at[s]),
          (dr_hbm.at[pl.ds(r0, CH)], dreal_v.at[s]),
      ]

    def b_pairs(s):
      ps = []
      for j in range(SPC):
        dst = pl.ds(j * L, L)
        ps.append((dt_hbm.at[didx_v.at[s, j]], drows_v.at[s, dst]))
        ps.append((t0_hbm.at[s0i_v.at[s, j]], s0rows_v.at[s, dst]))
        ps.append((t1_hbm.at[s1i_v.at[s, j]], s1rows_v.at[s, dst]))
        ps.append((sreal_hbm.at[bidx_v.at[s, j]], srrows_v.at[s, dst]))
      return ps

    def c_pairs(g, s):
      r0 = (c0 + g) * CH
      return [
          (s0rows_v.at[s], out_hbm.at[pl.ds(r0, CH), pl.ds(0, 32)]),
          (s1rows_v.at[s], out_hbm.at[pl.ds(r0, CH), pl.ds(32, 32)]),
          (srrows_v.at[s], out_hbm.at[pl.ds(r0, CH), pl.ds(64, 8)]),
          (drows_v.at[s], out_hbm.at[pl.ds(r0, CH), pl.ds(72, 32)]),
          (dreal_v.at[s], out_hbm.at[pl.ds(r0, CH), pl.ds(104, 8)]),
      ]

    def issue(pairs, sem):
      for src, dst in pairs:
        pltpu.async_copy(src, dst, sem)

    def drain(pairs, sem):
      for src, dst in pairs:
        pltpu.make_async_copy(src, dst, sem).wait()

    def step(g, s):
      drain(a_pairs(g, s), sem_st)
      issue(b_pairs(s), sem_g)
      drain(c_pairs(g - 1, 1 - s), sem_w)
      issue(a_pairs(g + 1, 1 - s), sem_st)
      drain(b_pairs(s), sem_g)
      issue(c_pairs(g, s), sem_w)

    issue(a_pairs(0, 0), sem_st)
    drain(a_pairs(0, 0), sem_st)
    issue(b_pairs(0), sem_g)
    issue(a_pairs(1, 1), sem_st)
    drain(b_pairs(0), sem_g)
    issue(c_pairs(0, 0), sem_w)

    def body(p, carry):
      g = 1 + 2 * p
      step(g, 1)
      step(g + 1, 0)
      return carry

    lax.fori_loop(0, (CH_PER_W - 2) // 2, body, 0)

    g_last = CH_PER_W - 1
    drain(a_pairs(g_last, 1), sem_st)
    issue(b_pairs(1), sem_g)
    drain(c_pairs(g_last - 1, 0), sem_w)
    drain(b_pairs(1), sem_g)
    issue(c_pairs(g_last, 1), sem_w)
    drain(c_pairs(g_last, 1), sem_w)

  return k(didx2d, s0r2d, s1r2d, bidx2d, sreal, dyn_real2d, t0, t1, dt)


def kernel(feat_static_cat, feat_static_real, feat_dynamic_cat,
           feat_dynamic_real, static_table0, static_table1, dyn_table0):
  didx2d = feat_dynamic_cat.astype(jnp.int32).reshape(N_L, L)
  s0r2d = jnp.repeat(feat_static_cat[:, 0].astype(jnp.int32),
                     T).reshape(N_L, L)
  s1r2d = jnp.repeat(feat_static_cat[:, 1].astype(jnp.int32),
                     T).reshape(N_L, L)
  bidx2d = jnp.repeat(jnp.arange(B, dtype=jnp.int32), T).reshape(N_L, L)
  dr2d = feat_dynamic_real.reshape(BT, 8)
  out = _sc_assemble(didx2d, s0r2d, s1r2d, bidx2d, feat_static_real, dr2d,
                     static_table0, static_table1, dyn_table0)
  return out.reshape(B, T, D_OUT)
